# baseline (speedup 1.0000x reference)
import functools

import jax
import jax.numpy as jnp
from jax import lax
from jax.experimental import pallas as pl
from jax.experimental.pallas import tpu as pltpu
from jax.experimental.pallas import tpu_sc as plsc

N = 10000
E = 160000
D = 256
H = 512
G = 128
FC = 128
CH = H // FC
NC, NS, L = 2, 16, 16

NP = 10240
EB = 112
EPT = E // NS
NBLK = 96
EPAD = NBLK * EB
NGRP = 4
GBLK = NBLK // NGRP
ACC_ROWS = NP
DUMMY = N
STRIPE = NP // NS
NB = 1024
NGRID = NP // NB

_mesh = plsc.VectorSubcoreMesh(core_axis_name="c", subcore_axis_name="s")

f32 = jnp.float32
i32 = jnp.int32


def _mm0_body(x_ref, w_ref, o0, o1, o2, o3):
    m = jnp.dot(x_ref[...], w_ref[...], preferred_element_type=f32,
                precision=lax.Precision.HIGHEST)
    for c, o in enumerate((o0, o1, o2, o3)):
        o[...] = m[:, c * FC:(c + 1) * FC]


def _mm0(x, w0):
    return pl.pallas_call(
        _mm0_body,
        grid=(NGRID,),
        in_specs=[
            pl.BlockSpec((NB, D), lambda i: (i, 0)),
            pl.BlockSpec((D, H), lambda i: (0, 0)),
        ],
        out_specs=[pl.BlockSpec((NB, FC), lambda i: (i, 0))] * CH,
        out_shape=[jax.ShapeDtypeStruct((NP, FC), f32)] * CH,
    )(x, w0)


def _mml_body(a0, a1, a2, a3, b_ref, w_ref, o0, o1, o2, o3):
    m = jnp.zeros((NB, H), f32)
    for c, a in enumerate((a0, a1, a2, a3)):
        h = jnp.maximum(a[...] + b_ref[c:c + 1, :], 0.0)
        m = m + jnp.dot(h, w_ref[c], preferred_element_type=f32,
                        precision=lax.Precision.HIGHEST)
    for c, o in enumerate((o0, o1, o2, o3)):
        o[...] = m[:, c * FC:(c + 1) * FC]


def _mml(aggs, b2d, w3d):
    return pl.pallas_call(
        _mml_body,
        grid=(NGRID,),
        in_specs=[pl.BlockSpec((NB, FC), lambda i: (i, 0))] * CH + [
            pl.BlockSpec((CH, FC), lambda i: (0, 0)),
            pl.BlockSpec((CH, FC, H), lambda i: (0, 0, 0)),
        ],
        out_specs=[pl.BlockSpec((NB, FC), lambda i: (i, 0))] * CH,
        out_shape=[jax.ShapeDtypeStruct((NP, FC), f32)] * CH,
    )(*aggs, b2d, w3d)


@functools.partial(
    pl.kernel,
    out_type=tuple(jax.ShapeDtypeStruct((NP, FC), f32) for _ in range(CH)),
    mesh=_mesh,
    scratch_types=[
        pltpu.VMEM((GBLK, EB), i32),
        pltpu.VMEM((GBLK, EB), i32),
        pltpu.VMEM((EB, FC), f32),
        pltpu.VMEM((EB, FC), f32),
        pltpu.VMEM((EB, FC), f32),
        pltpu.VMEM_SHARED((ACC_ROWS, FC), f32),
        pltpu.SemaphoreType.DMA,
        pltpu.SemaphoreType.DMA,
        pltpu.SemaphoreType.DMA,
    ],
)
def _sc_scatter(m0, m1, m2, m3, srcp, dstp, o0, o1, o2, o3,
                src_v, dst_v, ebuf0, ebuf1, ebuf2, acc, gsem0, gsem1, gsem2):
    c = lax.axis_index("c")
    s = lax.axis_index("s")
    ebuf = ebuf0
    ebufs = (ebuf0, ebuf1, ebuf2)
    gsems = (gsem0, gsem1, gsem2)

    ms = (m0, m1, m2, m3)
    outs = (o0, o1, o2, o3)

    def _edges(m_hbm):
        for grp in range(NGRP):
            pltpu.sync_copy(srcp.at[s, pl.ds(grp * GBLK, GBLK)], src_v)
            pltpu.sync_copy(dstp.at[s, pl.ds(grp * GBLK, GBLK)], dst_v)
            for par in range(3):
                pltpu.async_copy(m_hbm.at[src_v.at[par]], ebufs[par],
                                 gsems[par])

            def body(t, carry):
                for par in range(3):
                    j = 3 * t + par
                    pltpu.make_async_copy(m_hbm.at[src_v.at[j]], ebufs[par],
                                          gsems[par]).wait()
                    pltpu.sync_copy(ebufs[par], acc.at[dst_v.at[j]], add=True)

                    @pl.when(t < GBLK // 3 - 1)
                    def _():
                        pltpu.async_copy(m_hbm.at[src_v.at[j + 3]],
                                         ebufs[par], gsems[par])
                return carry
            lax.fori_loop(0, GBLK // 3, body, 0)

    def _writeback(out_hbm):
        pltpu.sync_copy(acc.at[pl.ds(s * STRIPE, STRIPE)],
                        out_hbm.at[pl.ds(s * STRIPE, STRIPE)])

    for q in range(2):
        def _zrow(zi, carry):
            for k in range(FC // L):
                ebuf[zi, k * L:(k + 1) * L] = jnp.zeros((L,), f32)
            return carry
        lax.fori_loop(0, EB, _zrow, 0)
        zoff = 0
        while zoff < STRIPE:
            zn = min(EB, STRIPE - zoff)
            pltpu.sync_copy(ebuf.at[pl.ds(0, zn)],
                            acc.at[pl.ds(s * STRIPE + zoff, zn)])
            zoff += zn
        plsc.subcore_barrier()

        @pl.when(c == 0)
        def _():
            _edges(ms[q])

        @pl.when(c == 1)
        def _():
            _edges(ms[2 + q])
        plsc.subcore_barrier()

        @pl.when(c == 0)
        def _():
            _writeback(outs[q])

        @pl.when(c == 1)
        def _():
            _writeback(outs[2 + q])
        plsc.subcore_barrier()


def _pool_body(a0, a1, a2, a3, b_ref, bb_ref, h0, h1, h2, h3,
               s0, s1, s2, s3, cnt):
    i = pl.program_id(0)
    gids = lax.broadcasted_iota(i32, (NB, G), 1).astype(f32)
    onehot = (bb_ref[...] == gids).astype(f32)
    houts = (h0, h1, h2, h3)
    souts = (s0, s1, s2, s3)
    for c, a in enumerate((a0, a1, a2, a3)):
        h = jnp.maximum(a[...] + b_ref[c:c + 1, :], 0.0)
        houts[c][...] = h
        contrib = lax.dot_general(onehot, h, (((0,), (0,)), ((), ())),
                                  preferred_element_type=f32,
                                  precision=lax.Precision.HIGHEST)

        @pl.when(i == 0)
        def _():
            souts[c][...] = contrib

        @pl.when(i > 0)
        def _():
            souts[c][...] = souts[c][...] + contrib
    ccontrib = lax.dot_general(onehot, jnp.ones((NB, FC), f32),
                               (((0,), (0,)), ((), ())),
                               preferred_element_type=f32,
                               precision=lax.Precision.HIGHEST)

    @pl.when(i == 0)
    def _():
        cnt[...] = ccontrib

    @pl.when(i > 0)
    def _():
        cnt[...] = cnt[...] + ccontrib


def _pool_sums(aggs, b2d, batchb):
    return pl.pallas_call(
        _pool_body,
        grid=(NGRID,),
        in_specs=[pl.BlockSpec((NB, FC), lambda i: (i, 0))] * CH + [
            pl.BlockSpec((CH, FC), lambda i: (0, 0)),
            pl.BlockSpec((NB, G), lambda i: (i, 0)),
        ],
        out_specs=[pl.BlockSpec((NB, FC), lambda i: (i, 0))] * CH
        + [pl.BlockSpec((G, FC), lambda i: (0, 0))] * CH
        + [pl.BlockSpec((G, FC), lambda i: (0, 0))],
        out_shape=[jax.ShapeDtypeStruct((NP, FC), f32)] * CH
        + [jax.ShapeDtypeStruct((G, FC), f32)] * CH
        + [jax.ShapeDtypeStruct((G, FC), f32)],
    )(*aggs, b2d, batchb)


NPT = NP // NS
BPAD = NPT + L
GP = G + 8


@functools.partial(
    pl.kernel,
    out_type=tuple(jax.ShapeDtypeStruct((G, FC), f32) for _ in range(CH)),
    mesh=_mesh,
    scratch_types=[
        pltpu.VMEM((NPT, FC), f32),
        pltpu.VMEM((BPAD,), i32),
        pltpu.VMEM((GP, FC), f32),
        pltpu.VMEM((8, FC), f32),
        pltpu.VMEM((8, FC), f32),
        pltpu.VMEM_SHARED((NS, G, FC), f32),
        pltpu.SemaphoreType.DMA,
    ],
)
def _sc_maxpool(h0, h1, h2, h3, batchp, o0, o1, o2, o3,
                hbuf, bbuf, lmax, cbuf, tbuf, spool, sem):
    c = lax.axis_index("c")
    s = lax.axis_index("s")
    pltpu.sync_copy(batchp.at[s], bbuf)
    hs = (h0, h1, h2, h3)
    outs = (o0, o1, o2, o3)
    base = s * 8

    for q in range(2):
        def _zrow(i, carry):
            for k in range(FC // L):
                lmax[i, k * L:(k + 1) * L] = jnp.zeros((L,), f32)
            return carry
        lax.fori_loop(0, GP, _zrow, 0)

        def _accum(h_hbm):
            pltpu.sync_copy(h_hbm.at[pl.ds(s * NPT, NPT)], hbuf)

            def body(i, carry):
                g = bbuf[pl.ds(i, L)][0]
                for k in range(FC // L):
                    sl = pl.ds(k * L, L)
                    lmax[g, sl] = jnp.maximum(lmax[g, sl], hbuf[i, sl])
                return carry
            lax.fori_loop(0, NPT, body, 0)

        @pl.when(c == 0)
        def _():
            _accum(hs[q])

        @pl.when(c == 1)
        def _():
            _accum(hs[2 + q])

        pltpu.sync_copy(lmax.at[pl.ds(0, G)], spool.at[s])
        plsc.subcore_barrier()

        for t in range(NS):
            pltpu.sync_copy(spool.at[t, pl.ds(base, 8)], tbuf)
            if t == 0:
                for r in range(8):
                    for k in range(FC // L):
                        sl = pl.ds(k * L, L)
                        cbuf[r, sl] = tbuf[r, sl]
            else:
                for r in range(8):
                    for k in range(FC // L):
                        sl = pl.ds(k * L, L)
                        cbuf[r, sl] = jnp.maximum(cbuf[r, sl], tbuf[r, sl])

        @pl.when(c == 0)
        def _():
            pltpu.sync_copy(cbuf, outs[q].at[pl.ds(base, 8)])

        @pl.when(c == 1)
        def _():
            pltpu.sync_copy(cbuf, outs[2 + q].at[pl.ds(base, 8)])
        plsc.subcore_barrier()


def _final_body(mx0, mx1, mx2, mx3, s0, s1, s2, s3, cnt_ref, w_ref, bb_ref,
                o_ref):
    cnt = jnp.maximum(cnt_ref[...], 1.0)
    tot = jnp.zeros((G, FC), f32)
    for cidx, (mx, sm) in enumerate(zip((mx0, mx1, mx2, mx3),
                                        (s0, s1, s2, s3))):
        w1 = w_ref[cidx:cidx + 1, :]
        w2 = w_ref[CH + cidx:CH + cidx + 1, :]
        tot = tot + mx[...] * w1 + (sm[...] / cnt) * w2
    out = jnp.sum(tot, axis=1, keepdims=True) + bb_ref[0, 0]
    o_ref[...] = jnp.broadcast_to(out, (G, FC))


def _final(mxs, sums, cnt, woutp, boutb):
    return pl.pallas_call(
        _final_body,
        in_specs=[pl.BlockSpec((G, FC), lambda: (0, 0))] * (2 * CH + 1) + [
            pl.BlockSpec((2 * CH, FC), lambda: (0, 0)),
            pl.BlockSpec((8, FC), lambda: (0, 0)),
        ],
        out_specs=pl.BlockSpec((G, FC), lambda: (0, 0)),
        out_shape=jax.ShapeDtypeStruct((G, FC), f32),
    )(*mxs, *sums, cnt, woutp, boutb)


def kernel(x, edge_index, batch, W0, b0, W1, b1, W2, b2, W3, b3, Wout, bout):
    src = edge_index[0]
    dst = edge_index[1]
    pad = EPAD - EPT
    srcp = jnp.concatenate(
        [src.reshape(NS, EPT), jnp.zeros((NS, pad), i32)], axis=1
    ).reshape(NS, NBLK, EB)
    dstp = jnp.concatenate(
        [dst.reshape(NS, EPT), jnp.full((NS, pad), DUMMY, i32)], axis=1
    ).reshape(NS, NBLK, EB)
    xp = jnp.concatenate([x, jnp.zeros((NP - N, D), f32)], axis=0)
    batch_pad = jnp.concatenate([batch, jnp.full((NP - N,), -1, i32)])
    batchb = jnp.broadcast_to(batch_pad[:, None], (NP, G)).astype(f32)
    batchg = jnp.concatenate([batch, jnp.full((NP - N,), G, i32)])
    batchp = jnp.concatenate(
        [batchg.reshape(NS, NPT), jnp.full((NS, BPAD - NPT), G, i32)], axis=1)

    w1_3d = W1.reshape(CH, FC, H)
    w2_3d = W2.reshape(CH, FC, H)
    w3_3d = W3.reshape(CH, FC, H)
    b0_2d = b0.reshape(CH, FC)
    b1_2d = b1.reshape(CH, FC)
    b2_2d = b2.reshape(CH, FC)
    b3_2d = b3.reshape(CH, FC)
    woutp = Wout[:, 0].reshape(2 * CH, FC)
    boutb = jnp.broadcast_to(bout.reshape(1, 1), (8, FC))

    m = _mm0(xp, W0)
    agg = _sc_scatter(*m, srcp, dstp)
    for b2d, w3d in ((b0_2d, w1_3d), (b1_2d, w2_3d), (b2_2d, w3_3d)):
        m = _mml(agg, b2d, w3d)
        agg = _sc_scatter(*m, srcp, dstp)

    pool = _pool_sums(agg, b3_2d, batchb)
    h3 = pool[:CH]
    sums = pool[CH:2 * CH]
    cnt = pool[2 * CH]
    mxs = _sc_maxpool(*h3, batchp)
    out2d = _final(mxs, sums, cnt, woutp, boutb)
    return out2d[:, :1]


# speedup vs baseline: 2.1965x; 2.1965x over previous
import functools

import jax
import jax.numpy as jnp
from jax import lax
from jax.experimental import pallas as pl
from jax.experimental.pallas import tpu as pltpu
from jax.experimental.pallas import tpu_sc as plsc

N = 10000
E = 160000
D = 256
H = 512
G = 128
FC = 128
CH = H // FC
NC, NS, L = 2, 16, 16

NP = 10240
EB = 128
EPT = E // NS
NBLK = 80
EPAD = NBLK * EB
NGRP = 2
GBLK = NBLK // NGRP
ACC_ROWS = NP
DUMMY = N
STRIPE = NP // NS
NB = 1024
NGRID = NP // NB

_mesh = plsc.VectorSubcoreMesh(core_axis_name="c", subcore_axis_name="s")

f32 = jnp.float32
i32 = jnp.int32


def _mm0_body(x_ref, w_ref, o0, o1, o2, o3):
    m = jnp.dot(x_ref[...], w_ref[...], preferred_element_type=f32,
                precision=lax.Precision.DEFAULT)
    for c, o in enumerate((o0, o1, o2, o3)):
        o[...] = m[:, c * FC:(c + 1) * FC]


def _mm0(x, w0):
    return pl.pallas_call(
        _mm0_body,
        grid=(NGRID,),
        in_specs=[
            pl.BlockSpec((NB, D), lambda i: (i, 0)),
            pl.BlockSpec((D, H), lambda i: (0, 0)),
        ],
        out_specs=[pl.BlockSpec((NB, FC), lambda i: (i, 0))] * CH,
        out_shape=[jax.ShapeDtypeStruct((NP, FC), f32)] * CH,
    )(x, w0)


def _mml_body(a0, a1, a2, a3, b_ref, w_ref, o0, o1, o2, o3):
    m = jnp.zeros((NB, H), f32)
    for c, a in enumerate((a0, a1, a2, a3)):
        h = jnp.maximum(a[...] + b_ref[c:c + 1, :], 0.0)
        m = m + jnp.dot(h, w_ref[c], preferred_element_type=f32,
                        precision=lax.Precision.DEFAULT)
    for c, o in enumerate((o0, o1, o2, o3)):
        o[...] = m[:, c * FC:(c + 1) * FC]


def _mml(aggs, b2d, w3d):
    return pl.pallas_call(
        _mml_body,
        grid=(NGRID,),
        in_specs=[pl.BlockSpec((NB, FC), lambda i: (i, 0))] * CH + [
            pl.BlockSpec((CH, FC), lambda i: (0, 0)),
            pl.BlockSpec((CH, FC, H), lambda i: (0, 0, 0)),
        ],
        out_specs=[pl.BlockSpec((NB, FC), lambda i: (i, 0))] * CH,
        out_shape=[jax.ShapeDtypeStruct((NP, FC), f32)] * CH,
    )(*aggs, b2d, w3d)


@functools.partial(
    pl.kernel,
    out_type=tuple(jax.ShapeDtypeStruct((NP, FC), f32) for _ in range(CH)),
    mesh=_mesh,
    scratch_types=[
        pltpu.VMEM((GBLK, EB), i32),
        pltpu.VMEM((GBLK, EB), i32),
        pltpu.VMEM((EB, FC), f32),
        pltpu.VMEM((EB, FC), f32),
        pltpu.VMEM_SHARED((ACC_ROWS, FC), f32),
        pltpu.SemaphoreType.DMA,
        pltpu.SemaphoreType.DMA,
    ],
)
def _sc_scatter(m0, m1, m2, m3, srcp, dstp, o0, o1, o2, o3,
                src_v, dst_v, ebuf0, ebuf1, acc, gsem0, gsem1):
    c = lax.axis_index("c")
    s = lax.axis_index("s")
    ebuf = ebuf0
    ebufs = (ebuf0, ebuf1)
    gsems = (gsem0, gsem1)

    ms = (m0, m1, m2, m3)
    outs = (o0, o1, o2, o3)

    def _edges(m_hbm):
        for grp in range(NGRP):
            pltpu.sync_copy(srcp.at[s, pl.ds(grp * GBLK, GBLK)], src_v)
            pltpu.sync_copy(dstp.at[s, pl.ds(grp * GBLK, GBLK)], dst_v)
            for par in range(2):
                pltpu.async_copy(m_hbm.at[src_v.at[par]], ebufs[par],
                                 gsems[par])

            def body(t, carry):
                for par in range(2):
                    j = 2 * t + par
                    pltpu.make_async_copy(m_hbm.at[src_v.at[j]], ebufs[par],
                                          gsems[par]).wait()
                    pltpu.sync_copy(ebufs[par], acc.at[dst_v.at[j]], add=True)

                    @pl.when(t < GBLK // 2 - 1)
                    def _():
                        pltpu.async_copy(m_hbm.at[src_v.at[j + 2]],
                                         ebufs[par], gsems[par])
                return carry
            lax.fori_loop(0, GBLK // 2, body, 0)

    def _writeback(out_hbm):
        pltpu.sync_copy(acc.at[pl.ds(s * STRIPE, STRIPE)],
                        out_hbm.at[pl.ds(s * STRIPE, STRIPE)])

    for q in range(2):
        def _zrow(zi, carry):
            for k in range(FC // L):
                ebuf[zi, k * L:(k + 1) * L] = jnp.zeros((L,), f32)
            return carry
        lax.fori_loop(0, EB, _zrow, 0)
        zoff = 0
        while zoff < STRIPE:
            zn = min(EB, STRIPE - zoff)
            pltpu.sync_copy(ebuf.at[pl.ds(0, zn)],
                            acc.at[pl.ds(s * STRIPE + zoff, zn)])
            zoff += zn
        plsc.subcore_barrier()

        @pl.when(c == 0)
        def _():
            _edges(ms[q])

        @pl.when(c == 1)
        def _():
            _edges(ms[2 + q])
        plsc.subcore_barrier()

        @pl.when(c == 0)
        def _():
            _writeback(outs[q])

        @pl.when(c == 1)
        def _():
            _writeback(outs[2 + q])
        plsc.subcore_barrier()


def _pool_body(a0, a1, a2, a3, b_ref, bb_ref, h0, h1, h2, h3,
               s0, s1, s2, s3, cnt):
    i = pl.program_id(0)
    gids = lax.broadcasted_iota(i32, (NB, G), 1).astype(f32)
    onehot = (bb_ref[...] == gids).astype(f32)
    houts = (h0, h1, h2, h3)
    souts = (s0, s1, s2, s3)
    for c, a in enumerate((a0, a1, a2, a3)):
        h = jnp.maximum(a[...] + b_ref[c:c + 1, :], 0.0)
        houts[c][...] = h
        contrib = lax.dot_general(onehot, h, (((0,), (0,)), ((), ())),
                                  preferred_element_type=f32,
                                  precision=lax.Precision.DEFAULT)

        @pl.when(i == 0)
        def _():
            souts[c][...] = contrib

        @pl.when(i > 0)
        def _():
            souts[c][...] = souts[c][...] + contrib
    ccontrib = lax.dot_general(onehot, jnp.ones((NB, FC), f32),
                               (((0,), (0,)), ((), ())),
                               preferred_element_type=f32,
                               precision=lax.Precision.DEFAULT)

    @pl.when(i == 0)
    def _():
        cnt[...] = ccontrib

    @pl.when(i > 0)
    def _():
        cnt[...] = cnt[...] + ccontrib


def _pool_sums(aggs, b2d, batchb):
    return pl.pallas_call(
        _pool_body,
        grid=(NGRID,),
        in_specs=[pl.BlockSpec((NB, FC), lambda i: (i, 0))] * CH + [
            pl.BlockSpec((CH, FC), lambda i: (0, 0)),
            pl.BlockSpec((NB, G), lambda i: (i, 0)),
        ],
        out_specs=[pl.BlockSpec((NB, FC), lambda i: (i, 0))] * CH
        + [pl.BlockSpec((G, FC), lambda i: (0, 0))] * CH
        + [pl.BlockSpec((G, FC), lambda i: (0, 0))],
        out_shape=[jax.ShapeDtypeStruct((NP, FC), f32)] * CH
        + [jax.ShapeDtypeStruct((G, FC), f32)] * CH
        + [jax.ShapeDtypeStruct((G, FC), f32)],
    )(*aggs, b2d, batchb)


NPT = NP // NS
BPAD = NPT + L
GP = G + 8


@functools.partial(
    pl.kernel,
    out_type=tuple(jax.ShapeDtypeStruct((G, FC), f32) for _ in range(CH)),
    mesh=_mesh,
    scratch_types=[
        pltpu.VMEM((NPT, FC), f32),
        pltpu.VMEM((BPAD,), i32),
        pltpu.VMEM((GP, FC), f32),
        pltpu.VMEM((8, FC), f32),
        pltpu.VMEM((8, FC), f32),
        pltpu.VMEM_SHARED((NS, G, FC), f32),
        pltpu.SemaphoreType.DMA,
    ],
)
def _sc_maxpool(h0, h1, h2, h3, batchp, o0, o1, o2, o3,
                hbuf, bbuf, lmax, cbuf, tbuf, spool, sem):
    c = lax.axis_index("c")
    s = lax.axis_index("s")
    pltpu.sync_copy(batchp.at[s], bbuf)
    hs = (h0, h1, h2, h3)
    outs = (o0, o1, o2, o3)
    base = s * 8

    for q in range(2):
        def _zrow(i, carry):
            for k in range(FC // L):
                lmax[i, k * L:(k + 1) * L] = jnp.zeros((L,), f32)
            return carry
        lax.fori_loop(0, GP, _zrow, 0)

        def _accum(h_hbm):
            pltpu.sync_copy(h_hbm.at[pl.ds(s * NPT, NPT)], hbuf)

            def body(i, carry):
                g = bbuf[pl.ds(i, L)][0]
                for k in range(FC // L):
                    sl = pl.ds(k * L, L)
                    lmax[g, sl] = jnp.maximum(lmax[g, sl], hbuf[i, sl])
                return carry
            lax.fori_loop(0, NPT, body, 0)

        @pl.when(c == 0)
        def _():
            _accum(hs[q])

        @pl.when(c == 1)
        def _():
            _accum(hs[2 + q])

        pltpu.sync_copy(lmax.at[pl.ds(0, G)], spool.at[s])
        plsc.subcore_barrier()

        for t in range(NS):
            pltpu.sync_copy(spool.at[t, pl.ds(base, 8)], tbuf)
            if t == 0:
                for r in range(8):
                    for k in range(FC // L):
                        sl = pl.ds(k * L, L)
                        cbuf[r, sl] = tbuf[r, sl]
            else:
                for r in range(8):
                    for k in range(FC // L):
                        sl = pl.ds(k * L, L)
                        cbuf[r, sl] = jnp.maximum(cbuf[r, sl], tbuf[r, sl])

        @pl.when(c == 0)
        def _():
            pltpu.sync_copy(cbuf, outs[q].at[pl.ds(base, 8)])

        @pl.when(c == 1)
        def _():
            pltpu.sync_copy(cbuf, outs[2 + q].at[pl.ds(base, 8)])
        plsc.subcore_barrier()


def _final_body(mx0, mx1, mx2, mx3, s0, s1, s2, s3, cnt_ref, w_ref, bb_ref,
                o_ref):
    cnt = jnp.maximum(cnt_ref[...], 1.0)
    tot = jnp.zeros((G, FC), f32)
    for cidx, (mx, sm) in enumerate(zip((mx0, mx1, mx2, mx3),
                                        (s0, s1, s2, s3))):
        w1 = w_ref[cidx:cidx + 1, :]
        w2 = w_ref[CH + cidx:CH + cidx + 1, :]
        tot = tot + mx[...] * w1 + (sm[...] / cnt) * w2
    out = jnp.sum(tot, axis=1, keepdims=True) + bb_ref[0, 0]
    o_ref[...] = jnp.broadcast_to(out, (G, FC))


def _final(mxs, sums, cnt, woutp, boutb):
    return pl.pallas_call(
        _final_body,
        in_specs=[pl.BlockSpec((G, FC), lambda: (0, 0))] * (2 * CH + 1) + [
            pl.BlockSpec((2 * CH, FC), lambda: (0, 0)),
            pl.BlockSpec((8, FC), lambda: (0, 0)),
        ],
        out_specs=pl.BlockSpec((G, FC), lambda: (0, 0)),
        out_shape=jax.ShapeDtypeStruct((G, FC), f32),
    )(*mxs, *sums, cnt, woutp, boutb)


def kernel(x, edge_index, batch, W0, b0, W1, b1, W2, b2, W3, b3, Wout, bout):
    src = edge_index[0]
    dst = edge_index[1]
    pad = EPAD - EPT
    srcp = jnp.concatenate(
        [src.reshape(NS, EPT), jnp.zeros((NS, pad), i32)], axis=1
    ).reshape(NS, NBLK, EB)
    dstp = jnp.concatenate(
        [dst.reshape(NS, EPT), jnp.full((NS, pad), DUMMY, i32)], axis=1
    ).reshape(NS, NBLK, EB)
    xp = jnp.concatenate([x, jnp.zeros((NP - N, D), f32)], axis=0)
    batch_pad = jnp.concatenate([batch, jnp.full((NP - N,), -1, i32)])
    batchb = jnp.broadcast_to(batch_pad[:, None], (NP, G)).astype(f32)
    batchg = jnp.concatenate([batch, jnp.full((NP - N,), G, i32)])
    batchp = jnp.concatenate(
        [batchg.reshape(NS, NPT), jnp.full((NS, BPAD - NPT), G, i32)], axis=1)

    w1_3d = W1.reshape(CH, FC, H)
    w2_3d = W2.reshape(CH, FC, H)
    w3_3d = W3.reshape(CH, FC, H)
    b0_2d = b0.reshape(CH, FC)
    b1_2d = b1.reshape(CH, FC)
    b2_2d = b2.reshape(CH, FC)
    b3_2d = b3.reshape(CH, FC)
    woutp = Wout[:, 0].reshape(2 * CH, FC)
    boutb = jnp.broadcast_to(bout.reshape(1, 1), (8, FC))

    m = _mm0(xp, W0)
    agg = _sc_scatter(*m, srcp, dstp)
    for b2d, w3d in ((b0_2d, w1_3d), (b1_2d, w2_3d), (b2_2d, w3_3d)):
        m = _mml(agg, b2d, w3d)
        agg = _sc_scatter(*m, srcp, dstp)

    pool = _pool_sums(agg, b3_2d, batchb)
    h3 = pool[:CH]
    sums = pool[CH:2 * CH]
    cnt = pool[2 * CH]
    mxs = _sc_maxpool(*h3, batchp)
    out2d = _final(mxs, sums, cnt, woutp, boutb)
    return out2d[:, :1]


# speedup vs baseline: 2.2002x; 1.0017x over previous
import functools

import jax
import jax.numpy as jnp
from jax import lax
from jax.experimental import pallas as pl
from jax.experimental.pallas import tpu as pltpu
from jax.experimental.pallas import tpu_sc as plsc

N = 10000
E = 160000
D = 256
H = 512
G = 128
FC = 128
CH = H // FC
NC, NS, L = 2, 16, 16

NP = 10240
EB = 128
EPT = E // NS
NBLK = 80
EPAD = NBLK * EB
NGRP = 2
GBLK = NBLK // NGRP
ACC_ROWS = NP
DUMMY = N
STRIPE = NP // NS
NB = 1024
NGRID = NP // NB

_mesh = plsc.VectorSubcoreMesh(core_axis_name="c", subcore_axis_name="s")

f32 = jnp.float32
i32 = jnp.int32


def _mm0_body(x_ref, w_ref, o0, o1, o2, o3):
    m = jnp.dot(x_ref[...], w_ref[...], preferred_element_type=f32,
                precision=lax.Precision.DEFAULT)
    for c, o in enumerate((o0, o1, o2, o3)):
        o[...] = m[:, c * FC:(c + 1) * FC]


def _mm0(x, w0):
    return pl.pallas_call(
        _mm0_body,
        grid=(NGRID,),
        in_specs=[
            pl.BlockSpec((NB, D), lambda i: (i, 0)),
            pl.BlockSpec((D, H), lambda i: (0, 0)),
        ],
        out_specs=[pl.BlockSpec((NB, FC), lambda i: (i, 0))] * CH,
        out_shape=[jax.ShapeDtypeStruct((NP, FC), f32)] * CH,
    )(x, w0)


def _mml_body(a0, a1, a2, a3, b_ref, w_ref, o0, o1, o2, o3):
    m = jnp.zeros((NB, H), f32)
    for c, a in enumerate((a0, a1, a2, a3)):
        h = jnp.maximum(a[...] + b_ref[c:c + 1, :], 0.0)
        m = m + jnp.dot(h, w_ref[c], preferred_element_type=f32,
                        precision=lax.Precision.DEFAULT)
    for c, o in enumerate((o0, o1, o2, o3)):
        o[...] = m[:, c * FC:(c + 1) * FC]


def _mml(aggs, b2d, w3d):
    return pl.pallas_call(
        _mml_body,
        grid=(NGRID,),
        in_specs=[pl.BlockSpec((NB, FC), lambda i: (i, 0))] * CH + [
            pl.BlockSpec((CH, FC), lambda i: (0, 0)),
            pl.BlockSpec((CH, FC, H), lambda i: (0, 0, 0)),
        ],
        out_specs=[pl.BlockSpec((NB, FC), lambda i: (i, 0))] * CH,
        out_shape=[jax.ShapeDtypeStruct((NP, FC), f32)] * CH,
    )(*aggs, b2d, w3d)


@functools.partial(
    pl.kernel,
    out_type=tuple(jax.ShapeDtypeStruct((NP, FC), f32) for _ in range(CH)),
    mesh=_mesh,
    scratch_types=[
        pltpu.VMEM((GBLK, EB), i32),
        pltpu.VMEM((GBLK, EB), i32),
        pltpu.VMEM((EB, FC), f32),
        pltpu.VMEM((EB, FC), f32),
        pltpu.VMEM_SHARED((ACC_ROWS, FC), f32),
        pltpu.SemaphoreType.DMA,
        pltpu.SemaphoreType.DMA,
    ],
)
def _sc_scatter(m0, m1, m2, m3, srcp, dstp, o0, o1, o2, o3,
                src_v, dst_v, ebuf0, ebuf1, acc, gsem0, gsem1):
    c = lax.axis_index("c")
    s = lax.axis_index("s")
    ebuf = ebuf0
    ebufs = (ebuf0, ebuf1)
    gsems = (gsem0, gsem1)

    ms = (m0, m1, m2, m3)
    outs = (o0, o1, o2, o3)

    def _edges(m_hbm):
        for grp in range(NGRP):
            pltpu.sync_copy(srcp.at[s, pl.ds(grp * GBLK, GBLK)], src_v)
            pltpu.sync_copy(dstp.at[s, pl.ds(grp * GBLK, GBLK)], dst_v)
            for par in range(2):
                pltpu.async_copy(m_hbm.at[src_v.at[par]], ebufs[par],
                                 gsems[par])

            def body(t, carry):
                for par in range(2):
                    j = 2 * t + par
                    pltpu.make_async_copy(m_hbm.at[src_v.at[j]], ebufs[par],
                                          gsems[par]).wait()
                    pltpu.sync_copy(ebufs[par], acc.at[dst_v.at[j]], add=True)

                    @pl.when(t < GBLK // 2 - 1)
                    def _():
                        pltpu.async_copy(m_hbm.at[src_v.at[j + 2]],
                                         ebufs[par], gsems[par])
                return carry
            lax.fori_loop(0, GBLK // 2, body, 0)

    def _writeback(out_hbm):
        pltpu.sync_copy(acc.at[pl.ds(s * STRIPE, STRIPE)],
                        out_hbm.at[pl.ds(s * STRIPE, STRIPE)])

    for q in range(2):
        def _zrow(zi, carry):
            for k in range(FC // L):
                ebuf[zi, k * L:(k + 1) * L] = jnp.zeros((L,), f32)
            return carry
        lax.fori_loop(0, EB, _zrow, 0)
        zoff = 0
        while zoff < STRIPE:
            zn = min(EB, STRIPE - zoff)
            pltpu.sync_copy(ebuf.at[pl.ds(0, zn)],
                            acc.at[pl.ds(s * STRIPE + zoff, zn)])
            zoff += zn
        plsc.subcore_barrier()

        @pl.when(c == 0)
        def _():
            _edges(ms[q])

        @pl.when(c == 1)
        def _():
            _edges(ms[2 + q])
        plsc.subcore_barrier()

        @pl.when(c == 0)
        def _():
            _writeback(outs[q])

        @pl.when(c == 1)
        def _():
            _writeback(outs[2 + q])
        plsc.subcore_barrier()


def _pool_body(a0, a1, a2, a3, b_ref, bb_ref,
               s0, s1, s2, s3, cnt):
    i = pl.program_id(0)
    gids = lax.broadcasted_iota(i32, (NB, G), 1).astype(f32)
    onehot = (bb_ref[...] == gids).astype(f32)
    souts = (s0, s1, s2, s3)
    for c, a in enumerate((a0, a1, a2, a3)):
        h = jnp.maximum(a[...] + b_ref[c:c + 1, :], 0.0)
        contrib = lax.dot_general(onehot, h, (((0,), (0,)), ((), ())),
                                  preferred_element_type=f32,
                                  precision=lax.Precision.DEFAULT)

        @pl.when(i == 0)
        def _():
            souts[c][...] = contrib

        @pl.when(i > 0)
        def _():
            souts[c][...] = souts[c][...] + contrib
    ccontrib = lax.dot_general(onehot, jnp.ones((NB, FC), f32),
                               (((0,), (0,)), ((), ())),
                               preferred_element_type=f32,
                               precision=lax.Precision.DEFAULT)

    @pl.when(i == 0)
    def _():
        cnt[...] = ccontrib

    @pl.when(i > 0)
    def _():
        cnt[...] = cnt[...] + ccontrib


def _pool_sums(aggs, b2d, batchb):
    return pl.pallas_call(
        _pool_body,
        grid=(NGRID,),
        in_specs=[pl.BlockSpec((NB, FC), lambda i: (i, 0))] * CH + [
            pl.BlockSpec((CH, FC), lambda i: (0, 0)),
            pl.BlockSpec((NB, G), lambda i: (i, 0)),
        ],
        out_specs=[pl.BlockSpec((G, FC), lambda i: (0, 0))] * CH
        + [pl.BlockSpec((G, FC), lambda i: (0, 0))],
        out_shape=[jax.ShapeDtypeStruct((G, FC), f32)] * CH
        + [jax.ShapeDtypeStruct((G, FC), f32)],
    )(*aggs, b2d, batchb)


NPT = NP // NS
BPAD = NPT + L
GP = G + 8


@functools.partial(
    pl.kernel,
    out_type=tuple(jax.ShapeDtypeStruct((G, FC), f32) for _ in range(CH)),
    mesh=_mesh,
    scratch_types=[
        pltpu.VMEM((NPT, FC), f32),
        pltpu.VMEM((BPAD,), i32),
        pltpu.VMEM((8, FC), f32),
        pltpu.VMEM((GP, FC), f32),
        pltpu.VMEM((8, FC), f32),
        pltpu.VMEM((8, FC), f32),
        pltpu.VMEM_SHARED((NS, G, FC), f32),
        pltpu.SemaphoreType.DMA,
    ],
)
def _sc_maxpool(h0, h1, h2, h3, b3c, batchp, o0, o1, o2, o3,
                hbuf, bbuf, bias_v, lmax, cbuf, tbuf, spool, sem):
    c = lax.axis_index("c")
    s = lax.axis_index("s")
    pltpu.sync_copy(batchp.at[s], bbuf)
    pltpu.sync_copy(b3c, bias_v)
    hs = (h0, h1, h2, h3)
    outs = (o0, o1, o2, o3)
    base = s * 8

    for q in range(2):
        def _zrow(i, carry):
            for k in range(FC // L):
                lmax[i, k * L:(k + 1) * L] = jnp.zeros((L,), f32)
            return carry
        lax.fori_loop(0, GP, _zrow, 0)

        def _accum(h_hbm, cc):
            pltpu.sync_copy(h_hbm.at[pl.ds(s * NPT, NPT)], hbuf)

            def body(i, carry):
                g = bbuf[pl.ds(i, L)][0]
                for k in range(FC // L):
                    sl = pl.ds(k * L, L)
                    lmax[g, sl] = jnp.maximum(lmax[g, sl],
                                              hbuf[i, sl] + bias_v[cc, sl])
                return carry
            lax.fori_loop(0, NPT, body, 0)

        @pl.when(c == 0)
        def _():
            _accum(hs[q], q)

        @pl.when(c == 1)
        def _():
            _accum(hs[2 + q], 2 + q)

        pltpu.sync_copy(lmax.at[pl.ds(0, G)], spool.at[s])
        plsc.subcore_barrier()

        for t in range(NS):
            pltpu.sync_copy(spool.at[t, pl.ds(base, 8)], tbuf)
            if t == 0:
                for r in range(8):
                    for k in range(FC // L):
                        sl = pl.ds(k * L, L)
                        cbuf[r, sl] = tbuf[r, sl]
            else:
                for r in range(8):
                    for k in range(FC // L):
                        sl = pl.ds(k * L, L)
                        cbuf[r, sl] = jnp.maximum(cbuf[r, sl], tbuf[r, sl])

        @pl.when(c == 0)
        def _():
            pltpu.sync_copy(cbuf, outs[q].at[pl.ds(base, 8)])

        @pl.when(c == 1)
        def _():
            pltpu.sync_copy(cbuf, outs[2 + q].at[pl.ds(base, 8)])
        plsc.subcore_barrier()


def _final_body(mx0, mx1, mx2, mx3, s0, s1, s2, s3, cnt_ref, w_ref, bb_ref,
                o_ref):
    cnt = jnp.maximum(cnt_ref[...], 1.0)
    tot = jnp.zeros((G, FC), f32)
    for cidx, (mx, sm) in enumerate(zip((mx0, mx1, mx2, mx3),
                                        (s0, s1, s2, s3))):
        w1 = w_ref[cidx:cidx + 1, :]
        w2 = w_ref[CH + cidx:CH + cidx + 1, :]
        tot = tot + mx[...] * w1 + (sm[...] / cnt) * w2
    out = jnp.sum(tot, axis=1, keepdims=True) + bb_ref[0, 0]
    o_ref[...] = jnp.broadcast_to(out, (G, FC))


def _final(mxs, sums, cnt, woutp, boutb):
    return pl.pallas_call(
        _final_body,
        in_specs=[pl.BlockSpec((G, FC), lambda: (0, 0))] * (2 * CH + 1) + [
            pl.BlockSpec((2 * CH, FC), lambda: (0, 0)),
            pl.BlockSpec((8, FC), lambda: (0, 0)),
        ],
        out_specs=pl.BlockSpec((G, FC), lambda: (0, 0)),
        out_shape=jax.ShapeDtypeStruct((G, FC), f32),
    )(*mxs, *sums, cnt, woutp, boutb)


def kernel(x, edge_index, batch, W0, b0, W1, b1, W2, b2, W3, b3, Wout, bout):
    src = edge_index[0]
    dst = edge_index[1]
    pad = EPAD - EPT
    srcp = jnp.concatenate(
        [src.reshape(NS, EPT), jnp.zeros((NS, pad), i32)], axis=1
    ).reshape(NS, NBLK, EB)
    dstp = jnp.concatenate(
        [dst.reshape(NS, EPT), jnp.full((NS, pad), DUMMY, i32)], axis=1
    ).reshape(NS, NBLK, EB)
    xp = jnp.concatenate([x, jnp.zeros((NP - N, D), f32)], axis=0)
    batch_pad = jnp.concatenate([batch, jnp.full((NP - N,), -1, i32)])
    batchb = jnp.broadcast_to(batch_pad[:, None], (NP, G)).astype(f32)
    batchg = jnp.concatenate([batch, jnp.full((NP - N,), G, i32)])
    batchp = jnp.concatenate(
        [batchg.reshape(NS, NPT), jnp.full((NS, BPAD - NPT), G, i32)], axis=1)

    w1_3d = W1.reshape(CH, FC, H)
    w2_3d = W2.reshape(CH, FC, H)
    w3_3d = W3.reshape(CH, FC, H)
    b0_2d = b0.reshape(CH, FC)
    b1_2d = b1.reshape(CH, FC)
    b2_2d = b2.reshape(CH, FC)
    b3_2d = b3.reshape(CH, FC)
    woutp = Wout[:, 0].reshape(2 * CH, FC)
    boutb = jnp.broadcast_to(bout.reshape(1, 1), (8, FC))

    m = _mm0(xp, W0)
    agg = _sc_scatter(*m, srcp, dstp)
    for b2d, w3d in ((b0_2d, w1_3d), (b1_2d, w2_3d), (b2_2d, w3_3d)):
        m = _mml(agg, b2d, w3d)
        agg = _sc_scatter(*m, srcp, dstp)

    b3c = jnp.pad(b3.reshape(CH, FC), ((0, 8 - CH), (0, 0)))
    pool = _pool_sums(agg, b3_2d, batchb)
    sums = pool[:CH]
    cnt = pool[CH]
    mxs = _sc_maxpool(*agg, b3c, batchp)
    out2d = _final(mxs, sums, cnt, woutp, boutb)
    return out2d[:, :1]
